# baseline (speedup 1.0000x reference)
import functools

import jax
import jax.numpy as jnp
from jax import lax
from jax.experimental import pallas as pl
from jax.experimental.pallas import tpu as pltpu
from jax.experimental.pallas import tpu_sc as plsc

N = 10000
DH = 128
DOUT = 64
E = 320000

L = 128
NC = 2
NS = 16
NW = NC * NS
PROWS = 160
A0 = 104
A1 = PROWS - A0
AMAX = max(A0, A1)
EPAD = NS * PROWS * L
SR = 79
NP = SR * L
ZR = NP // NS
SCRAP = N

_F32 = jnp.float32


def _sc_mesh():
    return plsc.VectorSubcoreMesh(
        core_axis_name="c", subcore_axis_name="s", num_cores=NC, num_subcores=NS
    )


@functools.partial(
    pl.kernel,
    out_type=(
        jax.ShapeDtypeStruct((NC, SR, L), _F32),
        jax.ShapeDtypeStruct((NC, SR, L), _F32),
    ),
    mesh=_sc_mesh(),
    scratch_types=[
        pltpu.VMEM((PROWS // 2, L), jnp.int32),
        pltpu.VMEM((SR, L), _F32),
        pltpu.VMEM((SR, L), _F32),
        pltpu.VMEM((SR,), jnp.int32),
        pltpu.VMEM_SHARED((SR, L), _F32),
        pltpu.VMEM_SHARED((SR, L), _F32),
    ],
    compiler_params=pltpu.CompilerParams(needs_layout_passes=False),
)
def _deg_kernel(packed_hbm, zsheet_hbm, rowids_hbm, dsrc_out, ddst_out,
                packed_v, asrc_v, adst_v, rowids_v, sh_src, sh_dst):
    cid = lax.axis_index("c")
    sid = lax.axis_index("s")
    base = sid * PROWS + cid * (PROWS // 2)
    pltpu.sync_copy(zsheet_hbm, asrc_v)
    pltpu.sync_copy(zsheet_hbm, adst_v)
    pltpu.sync_copy(packed_hbm.at[pl.ds(base, PROWS // 2)], packed_v)
    pltpu.sync_copy(rowids_hbm, rowids_v)

    @pl.when(sid == 0)
    def _():
        pltpu.sync_copy(zsheet_hbm, sh_src)
        pltpu.sync_copy(zsheet_hbm, sh_dst)

    ones = jnp.ones((16,), _F32)

    def step(j, carry):
        for k in range(8):
            v16 = packed_v[j, pl.ds(16 * k, 16)]
            s16 = lax.bitwise_and(v16, 0xFFFF)
            d16 = lax.shift_right_logical(v16, 16)
            plsc.addupdate_scatter(
                asrc_v,
                [lax.shift_right_logical(s16, 7), lax.bitwise_and(s16, 127)],
                ones)
            plsc.addupdate_scatter(
                adst_v,
                [lax.shift_right_logical(d16, 7), lax.bitwise_and(d16, 127)],
                ones)
        return carry

    lax.fori_loop(0, PROWS // 2, step, 0)
    plsc.subcore_barrier()
    pltpu.sync_copy(asrc_v, sh_src.at[rowids_v], add=True)
    pltpu.sync_copy(adst_v, sh_dst.at[rowids_v], add=True)
    plsc.subcore_barrier()

    @pl.when(sid == 0)
    def _():
        pltpu.sync_copy(sh_src, dsrc_out.at[cid])
        pltpu.sync_copy(sh_dst, ddst_out.at[cid])


@functools.partial(
    pl.kernel,
    out_type=jax.ShapeDtypeStruct((NC, NP, DH), _F32),
    mesh=_sc_mesh(),
    scratch_types=[
        pltpu.VMEM((AMAX, L), jnp.int32),
        pltpu.VMEM((AMAX, L), jnp.int32),
        pltpu.VMEM((L, DH), _F32),
        pltpu.VMEM_SHARED((NP, DH), _F32),
        pltpu.SemaphoreType.DMA,
    ],
)
def _agg_kernel(xs_hbm, src_hbm, dst_hbm, zrow_hbm, parts_out,
                src_v, dst_v, rows_v, agg_sh, sem):
    cid = lax.axis_index("c")
    sid = lax.axis_index("s")
    base0 = sid * PROWS
    pltpu.sync_copy(zrow_hbm, agg_sh.at[pl.ds(sid * ZR, ZR)])

    @pl.when(cid == 0)
    def _():
        pltpu.sync_copy(src_hbm.at[pl.ds(base0, A0)],
                        src_v.at[pl.ds(0, A0)])
        pltpu.sync_copy(dst_hbm.at[pl.ds(base0, A0)],
                        dst_v.at[pl.ds(0, A0)])

    @pl.when(cid == 1)
    def _():
        pltpu.sync_copy(src_hbm.at[pl.ds(base0 + A0, A1)],
                        src_v.at[pl.ds(0, A1)])
        pltpu.sync_copy(dst_hbm.at[pl.ds(base0 + A0, A1)],
                        dst_v.at[pl.ds(0, A1)])

    plsc.subcore_barrier()

    def run(count):
        def step(j, carry):
            pltpu.async_copy(xs_hbm.at[src_v.at[j]], rows_v, sem).wait()
            pltpu.sync_copy(rows_v, agg_sh.at[dst_v.at[j]], add=True)
            return carry

        lax.fori_loop(0, count, step, 0)

    @pl.when(cid == 0)
    def _():
        run(A0)

    @pl.when(cid == 1)
    def _():
        run(A1)

    plsc.subcore_barrier()
    pltpu.sync_copy(agg_sh.at[pl.ds(sid * ZR, ZR)],
                    parts_out.at[cid, pl.ds(sid * ZR, ZR)])


def _att_pool(x, wg, wp, bp):
    g = jnp.dot(x, wg, preferred_element_type=_F32)
    m = jnp.max(g, axis=0, keepdims=True)
    e = jnp.exp(g - m)
    s = jnp.sum(e, axis=0, keepdims=True)
    pooled = jnp.sum((e / s) * x, axis=0, keepdims=True)
    return jnp.dot(pooled, wp, preferred_element_type=_F32) + bp


def _prologue_body(h_ref, dsrc_ref, ddst_ref, wg_ref, wp_ref, bp_ref,
                   xs_ref, rso_ref, rsi_ref, h0_ref):
    dsrc = dsrc_ref[0] + dsrc_ref[1]
    ddst = ddst_ref[0] + ddst_ref[1]
    rso = lax.rsqrt(jnp.maximum(dsrc, 1.0))
    rsi = lax.rsqrt(jnp.maximum(ddst, 1.0))
    rso_ref[...] = rso
    rsi_ref[...] = rsi
    x = h_ref[...]
    xs_ref[pl.ds(0, N), :] = x * rso[:N]
    xs_ref[pl.ds(N, NP - N), :] = jnp.zeros((NP - N, DH), _F32)
    h0_ref[...] = _att_pool(x, wg_ref[...], wp_ref[...], bp_ref[...])


_prologue_call = pl.pallas_call(
    _prologue_body,
    out_shape=(
        jax.ShapeDtypeStruct((NP, DH), _F32),
        jax.ShapeDtypeStruct((NP, 1), _F32),
        jax.ShapeDtypeStruct((NP, 1), _F32),
        jax.ShapeDtypeStruct((1, DOUT), _F32),
    ),
)


def _layer_body(p_ref, rsi_ref, rso_ref, wc_ref, bc_ref, wg_ref, wp_ref,
                bp_ref, xs_ref, h_ref):
    agg = (p_ref[0] + p_ref[1]) * rsi_ref[...]
    x = jnp.maximum(
        jnp.dot(agg, wc_ref[...], preferred_element_type=_F32) + bc_ref[...],
        0.0)
    xs_ref[...] = x * rso_ref[...]
    h_ref[...] = _att_pool(x[:N], wg_ref[...], wp_ref[...], bp_ref[...])


_layer_call = pl.pallas_call(
    _layer_body,
    out_shape=(
        jax.ShapeDtypeStruct((NP, DH), _F32),
        jax.ShapeDtypeStruct((1, DOUT), _F32),
    ),
)


def _final_body(p_ref, rsi_ref, wc_ref, bc_ref, wg_ref, wp_ref, bp_ref,
                h0_ref, h1_ref, h2_ref, out_ref):
    agg = (p_ref[0] + p_ref[1]) * rsi_ref[...]
    x = jnp.maximum(
        jnp.dot(agg, wc_ref[...], preferred_element_type=_F32) + bc_ref[...],
        0.0)
    h3 = _att_pool(x[:N], wg_ref[...], wp_ref[...], bp_ref[...])
    out_ref[...] = (h0_ref[...] + h1_ref[...] + h2_ref[...] + h3) * 0.25


_final_call = pl.pallas_call(
    _final_body,
    out_shape=jax.ShapeDtypeStruct((1, DOUT), _F32),
)


def kernel(h, edge_index, Wc0, bc0, Wc1, bc1, Wc2, bc2, Wg0, bg0, Wg1, bg1,
           Wg2, bg2, Wg3, bg3, Wp0, bp0, Wp1, bp1, Wp2, bp2, Wcls, bcls):
    del bg0, bg1, bg2, bg3
    packed = jnp.bitwise_or(edge_index[0],
                            jnp.left_shift(edge_index[1], 16))
    pad = jnp.full((EPAD - E,), SCRAP | (SCRAP << 16), jnp.int32)
    packed_r = jnp.concatenate([packed, pad]).reshape(NS * PROWS, L)
    zsheet = jnp.zeros((SR, L), _F32)
    zrow = jnp.zeros((ZR, DH), _F32)
    rowids = jnp.arange(SR, dtype=jnp.int32)

    src_r = jnp.concatenate(
        [edge_index[0], jnp.full((EPAD - E,), SCRAP, jnp.int32)]
    ).reshape(NS * PROWS, L)
    dst_r = jnp.concatenate(
        [edge_index[1], jnp.full((EPAD - E,), SCRAP, jnp.int32)]
    ).reshape(NS * PROWS, L)
    dsrc_p, ddst_p = _deg_kernel(packed_r, zsheet, rowids)
    xs, rso, rsi, h0 = _prologue_call(
        h, dsrc_p.reshape(NC, NP, 1), ddst_p.reshape(NC, NP, 1),
        Wg0, Wp0, bp0.reshape(1, DOUT))

    parts = _agg_kernel(xs, src_r, dst_r, zrow)
    xs, h1 = _layer_call(parts, rsi, rso, Wc0, bc0.reshape(1, DH), Wg1,
                         Wp1, bp1.reshape(1, DOUT))
    parts = _agg_kernel(xs, src_r, dst_r, zrow)
    xs, h2 = _layer_call(parts, rsi, rso, Wc1, bc1.reshape(1, DH), Wg2,
                         Wp2, bp2.reshape(1, DOUT))
    parts = _agg_kernel(xs, src_r, dst_r, zrow)
    return _final_call(parts, rsi, Wc2, bc2.reshape(1, DH), Wg3, Wcls,
                       bcls.reshape(1, DOUT), h0, h1, h2)


# speedup vs baseline: 3.0756x; 3.0756x over previous
import functools

import jax
import jax.numpy as jnp
from jax import lax
from jax.experimental import pallas as pl
from jax.experimental.pallas import tpu as pltpu
from jax.experimental.pallas import tpu_sc as plsc

N = 10000
DH = 128
DOUT = 64
E = 320000

L = 128
NC = 2
NS = 16
NW = NC * NS
PROWS = 160
A0 = 80
A1 = PROWS - A0
AMAX = max(A0, A1)
EPAD = NS * PROWS * L
SR = 79
NP = SR * L
ZR = NP // NS
SCRAP = N

_F32 = jnp.float32


def _sc_mesh():
    return plsc.VectorSubcoreMesh(
        core_axis_name="c", subcore_axis_name="s", num_cores=NC, num_subcores=NS
    )


@functools.partial(
    pl.kernel,
    out_type=(
        jax.ShapeDtypeStruct((NC, SR, L), _F32),
        jax.ShapeDtypeStruct((NC, SR, L), _F32),
    ),
    mesh=_sc_mesh(),
    scratch_types=[
        pltpu.VMEM((PROWS // 2, L), jnp.int32),
        pltpu.VMEM((SR, L), _F32),
        pltpu.VMEM((SR, L), _F32),
        pltpu.VMEM((SR,), jnp.int32),
        pltpu.VMEM_SHARED((SR, L), _F32),
        pltpu.VMEM_SHARED((SR, L), _F32),
    ],
    compiler_params=pltpu.CompilerParams(needs_layout_passes=False),
)
def _deg_kernel(packed_hbm, zsheet_hbm, rowids_hbm, dsrc_out, ddst_out,
                packed_v, asrc_v, adst_v, rowids_v, sh_src, sh_dst):
    cid = lax.axis_index("c")
    sid = lax.axis_index("s")
    base = sid * PROWS + cid * (PROWS // 2)
    pltpu.sync_copy(zsheet_hbm, asrc_v)
    pltpu.sync_copy(zsheet_hbm, adst_v)
    pltpu.sync_copy(packed_hbm.at[pl.ds(base, PROWS // 2)], packed_v)
    pltpu.sync_copy(rowids_hbm, rowids_v)

    @pl.when(sid == 0)
    def _():
        pltpu.sync_copy(zsheet_hbm, sh_src)
        pltpu.sync_copy(zsheet_hbm, sh_dst)

    ones = jnp.ones((16,), _F32)

    def step(j, carry):
        for k in range(8):
            v16 = packed_v[j, pl.ds(16 * k, 16)]
            s16 = lax.bitwise_and(v16, 0xFFFF)
            d16 = lax.shift_right_logical(v16, 16)
            plsc.addupdate_scatter(
                asrc_v,
                [lax.shift_right_logical(s16, 7), lax.bitwise_and(s16, 127)],
                ones)
            plsc.addupdate_scatter(
                adst_v,
                [lax.shift_right_logical(d16, 7), lax.bitwise_and(d16, 127)],
                ones)
        return carry

    lax.fori_loop(0, PROWS // 2, step, 0)
    plsc.subcore_barrier()
    pltpu.sync_copy(asrc_v, sh_src.at[rowids_v], add=True)
    pltpu.sync_copy(adst_v, sh_dst.at[rowids_v], add=True)
    plsc.subcore_barrier()

    @pl.when(sid == 0)
    def _():
        pltpu.sync_copy(sh_src, dsrc_out.at[cid])
        pltpu.sync_copy(sh_dst, ddst_out.at[cid])


@functools.partial(
    pl.kernel,
    out_type=jax.ShapeDtypeStruct((NC, NP, DH), _F32),
    mesh=_sc_mesh(),
    scratch_types=[
        pltpu.VMEM((AMAX, L), jnp.int32),
        pltpu.VMEM((2, L), jnp.int32),
        pltpu.VMEM((2, L), jnp.int32),
        pltpu.VMEM((L, DH), _F32),
        pltpu.VMEM((L, DH), _F32),
        pltpu.VMEM_SHARED((NP, DH), _F32),
        pltpu.SemaphoreType.DMA,
        pltpu.SemaphoreType.DMA,
    ],
    compiler_params=pltpu.CompilerParams(needs_layout_passes=False),
)
def _agg_kernel(xs_hbm, packed_hbm, zrow_hbm, parts_out,
                packed_v, sidx_v, didx_v, rows0_v, rows1_v, agg_sh,
                sem0, sem1):
    cid = lax.axis_index("c")
    sid = lax.axis_index("s")
    base0 = sid * PROWS
    pltpu.sync_copy(zrow_hbm, agg_sh.at[pl.ds(sid * ZR, ZR)])

    @pl.when(cid == 0)
    def _():
        pltpu.sync_copy(packed_hbm.at[pl.ds(base0, A0)],
                        packed_v.at[pl.ds(0, A0)])

    @pl.when(cid == 1)
    def _():
        pltpu.sync_copy(packed_hbm.at[pl.ds(base0 + A0, A1)],
                        packed_v.at[pl.ds(0, A1)])

    plsc.subcore_barrier()

    def unpack(j, slot):
        for k in range(8):
            v16 = packed_v[j, pl.ds(16 * k, 16)]
            sidx_v.at[slot][pl.ds(16 * k, 16)] = lax.bitwise_and(v16, 0xFFFF)
            didx_v.at[slot][pl.ds(16 * k, 16)] = lax.shift_right_logical(
                v16, 16)

    def gather(slot, rows_v, sem):
        pltpu.async_copy(xs_hbm.at[sidx_v.at[slot]], rows_v, sem)

    def gwait(rows_v, sem):
        pltpu.make_async_copy(xs_hbm.at[sidx_v.at[0]], rows_v, sem).wait()

    def scatter(slot, rows_v):
        pltpu.sync_copy(rows_v, agg_sh.at[didx_v.at[slot]], add=True)

    def run(count):
        unpack(0, 0)
        gather(0, rows0_v, sem0)

        def step(t, carry):
            j0 = 2 * t
            gwait(rows0_v, sem0)
            unpack(j0 + 1, 1)
            gather(1, rows1_v, sem1)
            scatter(0, rows0_v)
            gwait(rows1_v, sem1)
            unpack(j0 + 2, 0)
            gather(0, rows0_v, sem0)
            scatter(1, rows1_v)
            return carry

        lax.fori_loop(0, count // 2 - 1, step, 0)
        gwait(rows0_v, sem0)
        unpack(count - 1, 1)
        gather(1, rows1_v, sem1)
        scatter(0, rows0_v)
        gwait(rows1_v, sem1)
        scatter(1, rows1_v)

    @pl.when(cid == 0)
    def _():
        run(A0)

    @pl.when(cid == 1)
    def _():
        run(A1)

    plsc.subcore_barrier()
    pltpu.sync_copy(agg_sh.at[pl.ds(sid * ZR, ZR)],
                    parts_out.at[cid, pl.ds(sid * ZR, ZR)])


def _att_pool(x, wg, wp, bp):
    g = jnp.dot(x, wg, preferred_element_type=_F32)
    m = jnp.max(g, axis=0, keepdims=True)
    e = jnp.exp(g - m)
    s = jnp.sum(e, axis=0, keepdims=True)
    pooled = jnp.sum((e / s) * x, axis=0, keepdims=True)
    return jnp.dot(pooled, wp, preferred_element_type=_F32) + bp


def _prologue_body(h_ref, dsrc_ref, ddst_ref, wg_ref, wp_ref, bp_ref,
                   xs_ref, rso_ref, rsi_ref, h0_ref):
    dsrc = dsrc_ref[0] + dsrc_ref[1]
    ddst = ddst_ref[0] + ddst_ref[1]
    rso = lax.rsqrt(jnp.maximum(dsrc, 1.0))
    rsi = lax.rsqrt(jnp.maximum(ddst, 1.0))
    rso_ref[...] = rso
    rsi_ref[...] = rsi
    x = h_ref[...]
    xs_ref[pl.ds(0, N), :] = x * rso[:N]
    xs_ref[pl.ds(N, NP - N), :] = jnp.zeros((NP - N, DH), _F32)
    h0_ref[...] = _att_pool(x, wg_ref[...], wp_ref[...], bp_ref[...])


_prologue_call = pl.pallas_call(
    _prologue_body,
    out_shape=(
        jax.ShapeDtypeStruct((NP, DH), _F32),
        jax.ShapeDtypeStruct((NP, 1), _F32),
        jax.ShapeDtypeStruct((NP, 1), _F32),
        jax.ShapeDtypeStruct((1, DOUT), _F32),
    ),
)


def _layer_body(p_ref, rsi_ref, rso_ref, wc_ref, bc_ref, wg_ref, wp_ref,
                bp_ref, xs_ref, h_ref):
    agg = (p_ref[0] + p_ref[1]) * rsi_ref[...]
    x = jnp.maximum(
        jnp.dot(agg, wc_ref[...], preferred_element_type=_F32) + bc_ref[...],
        0.0)
    xs_ref[...] = x * rso_ref[...]
    h_ref[...] = _att_pool(x[:N], wg_ref[...], wp_ref[...], bp_ref[...])


_layer_call = pl.pallas_call(
    _layer_body,
    out_shape=(
        jax.ShapeDtypeStruct((NP, DH), _F32),
        jax.ShapeDtypeStruct((1, DOUT), _F32),
    ),
)


def _final_body(p_ref, rsi_ref, wc_ref, bc_ref, wg_ref, wp_ref, bp_ref,
                h0_ref, h1_ref, h2_ref, out_ref):
    agg = (p_ref[0] + p_ref[1]) * rsi_ref[...]
    x = jnp.maximum(
        jnp.dot(agg, wc_ref[...], preferred_element_type=_F32) + bc_ref[...],
        0.0)
    h3 = _att_pool(x[:N], wg_ref[...], wp_ref[...], bp_ref[...])
    out_ref[...] = (h0_ref[...] + h1_ref[...] + h2_ref[...] + h3) * 0.25


_final_call = pl.pallas_call(
    _final_body,
    out_shape=jax.ShapeDtypeStruct((1, DOUT), _F32),
)


def kernel(h, edge_index, Wc0, bc0, Wc1, bc1, Wc2, bc2, Wg0, bg0, Wg1, bg1,
           Wg2, bg2, Wg3, bg3, Wp0, bp0, Wp1, bp1, Wp2, bp2, Wcls, bcls):
    del bg0, bg1, bg2, bg3
    packed = jnp.bitwise_or(edge_index[0],
                            jnp.left_shift(edge_index[1], 16))
    scrap = SCRAP + jnp.arange(EPAD - E, dtype=jnp.int32) % (NP - N)
    pad = jnp.bitwise_or(scrap, jnp.left_shift(scrap, 16))
    packed_r = jnp.concatenate([packed, pad]).reshape(NS * PROWS, L)
    zsheet = jnp.zeros((SR, L), _F32)
    zrow = jnp.zeros((ZR, DH), _F32)
    rowids = jnp.arange(SR, dtype=jnp.int32)

    dsrc_p, ddst_p = _deg_kernel(packed_r, zsheet, rowids)
    xs, rso, rsi, h0 = _prologue_call(
        h, dsrc_p.reshape(NC, NP, 1), ddst_p.reshape(NC, NP, 1),
        Wg0, Wp0, bp0.reshape(1, DOUT))

    parts = _agg_kernel(xs, packed_r, zrow)
    xs, h1 = _layer_call(parts, rsi, rso, Wc0, bc0.reshape(1, DH), Wg1,
                         Wp1, bp1.reshape(1, DOUT))
    parts = _agg_kernel(xs, packed_r, zrow)
    xs, h2 = _layer_call(parts, rsi, rso, Wc1, bc1.reshape(1, DH), Wg2,
                         Wp2, bp2.reshape(1, DOUT))
    parts = _agg_kernel(xs, packed_r, zrow)
    return _final_call(parts, rsi, Wc2, bc2.reshape(1, DH), Wg3, Wcls,
                       bcls.reshape(1, DOUT), h0, h1, h2)
